# baseline (speedup 1.0000x reference)
import functools

import jax
import jax.numpy as jnp
from jax import lax
from jax.experimental import pallas as pl
from jax.experimental.pallas import tpu as pltpu
from jax.experimental.pallas import tpu_sc as plsc

NUM_ROWS = 1_000_000
DIM = 32
BATCH = 16384

_info = plsc.get_sparse_core_info()
_NC, _NS = _info.num_cores, _info.num_subcores
_NW = _NC * _NS
_TC_FULL = NUM_ROWS // 128
_TAIL0 = _TC_FULL * 128
_NTAIL = NUM_ROWS - _TAIL0
_WCOLS = 8
_WROWS = _WCOLS * 128
_NBINS = 32
_CAP = 64
_ROWS_OUT = BATCH + _NW
_BATCHR = 128
_FLUSH_AT = _BATCHR - 16

_BASE_TC = _TC_FULL // _NW
_EXTRA = _TC_FULL - _BASE_TC * _NW


def _c0(w):
    return _BASE_TC * w + jnp.minimum(w, _EXTRA)


@functools.partial(
    pl.kernel,
    mesh=plsc.VectorSubcoreMesh(core_axis_name="c", subcore_axis_name="s"),
    out_type=jax.ShapeDtypeStruct((_ROWS_OUT, 128), jnp.float32),
    scratch_types=[
        pltpu.VMEM((BATCH,), jnp.int32),
        pltpu.VMEM((BATCH + 16,), jnp.int32),
        pltpu.VMEM((_NBINS * _CAP,), jnp.int32),
        pltpu.VMEM((_NBINS,), jnp.int32),
        pltpu.VMEM((2, 4, 8, _WROWS), jnp.float32),
        pltpu.VMEM((_NTAIL, 128), jnp.float32),
        pltpu.VMEM((_BATCHR, 128), jnp.float32),
        pltpu.VMEM((_BATCHR,), jnp.int32),
        pltpu.SemaphoreType.DMA,
        pltpu.SemaphoreType.DMA,
    ],
    compiler_params=pltpu.CompilerParams(
        use_tc_tiling_on_sc=True, needs_layout_passes=False),
)
def _sweep_kernel(idx_hbm, tab_hbm, tail_hbm, out_hbm,
                  idxbuf, mk_v, bk_v, bc_v, win_v, tail_v, rows_v, kb_v,
                  sem, osem):
    wid = lax.axis_index("s") * _NC + lax.axis_index("c")
    iota = jax.lax.iota(jnp.int32, 16)
    m_idx = iota & 7
    j_a = jnp.where(iota < 8, 0, 1)
    j_b = j_a + 2
    zeros16 = jnp.zeros((16,), jnp.int32)

    c0 = _c0(wid)
    c1 = jnp.where(wid == _NW - 1, jnp.int32(_TC_FULL), _c0(wid + 1))
    lo = c0 * 128
    hi = jnp.where(wid == _NW - 1, jnp.int32(NUM_ROWS), c1 * 128)
    kdump = jnp.int32(BATCH) + wid

    pltpu.sync_copy(idx_hbm, idxbuf)
    pltpu.sync_copy(tail_hbm, tail_v)

    bc_v[pl.ds(0, 16)] = zeros16
    bc_v[pl.ds(16, 16)] = zeros16
    for b in range(_NBINS * _CAP // 16):
        bk_v[pl.ds(b * 16, 16)] = zeros16

    for j_big in range(4):
        pltpu.async_copy(
            tab_hbm.at[pl.ds(j_big * 8, 8), pl.ds(c0 * 128, _WROWS)],
            win_v.at[0, j_big], sem)

    def scan_body(blk, cnt):
        def grp(g, cnt):
            p = blk * 128 + g * 16
            ivec = idxbuf[pl.ds(p, 16)]
            msk = (ivec >= lo) & (ivec < hi)
            nm = plsc.all_reduce_population_count(msk)[0]

            def do(cnt):
                mi32 = msk.astype(jnp.int32)
                pos = cnt + plsc.cumsum(mi32) - mi32
                plsc.store_scatter(mk_v, [pos], p + iota, mask=msk)
                wv = jnp.clip((ivec - lo) >> 10, 0, _NBINS - 1)
                wv = jnp.where(ivec >= _TAIL0, _NBINS - 1, wv)
                rank, lastm = plsc.scan_count(wv, mask=msk)
                old = plsc.load_gather(bc_v, [wv])
                bpos = old + rank - 1
                plsc.store_scatter(
                    bk_v, [wv * _CAP + jnp.minimum(bpos, _CAP - 1)],
                    p + iota, mask=msk & (bpos < _CAP))
                plsc.store_scatter(bc_v, [wv], old + rank, mask=msk & lastm)
                return cnt + nm
            return lax.cond(nm > 0, do, lambda c: c, cnt)
        for g in range(8):
            cnt = grp(g, cnt)
        return cnt
    cnt = lax.fori_loop(0, BATCH // 128, scan_body, jnp.int32(0))
    mk_v[pl.ds(cnt, 16)] = zeros16

    def prefill():
        for b in range(_BATCHR // 16):
            kb_v[pl.ds(b * 16, 16)] = jnp.full((16,), kdump, jnp.int32)
    prefill()

    def flush():
        pltpu.async_copy(rows_v, out_hbm.at[kb_v], osem).wait()
        prefill()

    def extract_block(slot, kvec, valid, wlo, is_tail, win_ref=None):
        ivec = plsc.load_gather(idxbuf, [kvec])
        i_sel = jnp.where(valid, ivec, wlo)
        k_sel = jnp.where(valid, kvec, kdump)
        cl = i_sel - wlo
        slot_vec = slot + iota
        for j in range(DIM):
            j_spl = jnp.full((16,), j, jnp.int32)
            if is_tail:
                g = plsc.load_gather(tail_v, [cl, j_spl])
            else:
                g = plsc.load_gather(
                    win_ref,
                    [jnp.full((16,), j // 8, jnp.int32),
                     jnp.full((16,), j % 8, jnp.int32), cl])
            plsc.store_scatter(rows_v, [slot_vec, j_spl], g)
        kb_v[pl.ds(slot, 16)] = k_sel
        new_slot = slot + 16

        @pl.when(new_slot >= _BATCHR)
        def _():
            flush()
        return jnp.where(new_slot >= _BATCHR, 0, new_slot)

    def bucket_pass(slot, jw, wlo, bcnt, is_tail, win_ref):
        def grp_body(gg, slot):
            kvec = bk_v[pl.ds(jw * _CAP + gg * 16, 16)]
            valid = gg * 16 + iota < bcnt
            return extract_block(slot, kvec, valid, wlo, is_tail, win_ref)
        return lax.fori_loop(0, (bcnt + 15) // 16, grp_body, slot)

    def rescan_pass(slot, wlo, whi, is_tail, win_ref):
        def grp_body(g, slot):
            base = g * 16
            kvec = mk_v[pl.ds(base, 16)]
            ivec = plsc.load_gather(idxbuf, [kvec])
            valid = (base + iota < cnt) & (ivec >= wlo) & (ivec < whi)
            nmatch = plsc.all_reduce_population_count(valid)[0]
            return lax.cond(
                nmatch > 0,
                lambda s: extract_block(s, kvec, valid, wlo, is_tail, win_ref),
                lambda s: s, slot)
        return lax.fori_loop(0, (cnt + 15) // 16, grp_body, slot)

    def process_bin(slot, jw, wlo, whi, is_tail, win_ref):
        bcnt = plsc.load_gather(bc_v, [jnp.full((16,), jw, jnp.int32)])[0]
        return lax.cond(
            bcnt > _CAP,
            lambda s: rescan_pass(s, wlo, whi, is_tail, win_ref),
            lambda s: bucket_pass(s, jw, wlo, bcnt, is_tail, win_ref), slot)

    nwin = (c1 - c0 + _WCOLS - 1) // _WCOLS

    def wc0_of(jw):
        return jnp.minimum(c0 + jw * _WCOLS, c1 - _WCOLS)

    def issue(jw, buf):
        wc0 = wc0_of(jw)
        for j_big in range(4):
            pltpu.async_copy(
                tab_hbm.at[pl.ds(j_big * 8, 8), pl.ds(wc0 * 128, _WROWS)],
                win_v.at[buf, j_big], sem)

    def wait_win(buf):
        for j_big in range(4):
            pltpu.make_async_copy(
                tab_hbm.at[pl.ds(0, 8), pl.ds(0, _WROWS)],
                win_v.at[buf, j_big], sem).wait()

    def win_body(jw, slot):
        buf = jw & 1
        wait_win(buf)

        @pl.when(jw + 1 < nwin)
        def _():
            issue(jw + 1, buf ^ 1)
        wc0 = wc0_of(jw)
        return process_bin(slot, jw, wc0 * 128, (wc0 + _WCOLS) * 128, False,
                           win_v.at[buf])

    slot = lax.fori_loop(0, nwin, win_body, jnp.int32(0))

    slot = process_bin(slot, jnp.int32(_NBINS - 1), jnp.int32(_TAIL0),
                       jnp.int32(NUM_ROWS), True, win_v.at[0])

    flush()


def kernel(idx_list, table):
    idx = jnp.asarray(idx_list, jnp.int32)
    tab_t = table.T
    tail = jnp.pad(table[_TAIL0:], ((0, 0), (0, 128 - DIM)))
    out_raw = _sweep_kernel(idx, tab_t, tail)
    return out_raw[:BATCH, :DIM].reshape(1, BATCH, DIM)


# speedup vs baseline: 1.0688x; 1.0688x over previous
import functools

import jax
import jax.numpy as jnp
from jax import lax
from jax.experimental import pallas as pl
from jax.experimental.pallas import tpu as pltpu
from jax.experimental.pallas import tpu_sc as plsc

NUM_ROWS = 1_000_000
DIM = 32
BATCH = 16384

_info = plsc.get_sparse_core_info()
_NC, _NS = _info.num_cores, _info.num_subcores
_NW = _NC * _NS
_TC_FULL = NUM_ROWS // 128
_TAIL0 = _TC_FULL * 128
_NTAIL = NUM_ROWS - _TAIL0
_WCOLS = 8
_WROWS = _WCOLS * 128
_NBINS = 32
_CAP = 64
_ROWS_OUT = BATCH + _NW
_BATCHR = 128
_FLUSH_AT = _BATCHR - 16

_BASE_TC = _TC_FULL // _NW
_EXTRA = _TC_FULL - _BASE_TC * _NW


def _c0(w):
    return _BASE_TC * w + jnp.minimum(w, _EXTRA)


@functools.partial(
    pl.kernel,
    mesh=plsc.VectorSubcoreMesh(core_axis_name="c", subcore_axis_name="s"),
    out_type=jax.ShapeDtypeStruct((_ROWS_OUT, 128), jnp.float32),
    scratch_types=[
        pltpu.VMEM((BATCH,), jnp.int32),
        pltpu.VMEM((_NBINS * _CAP,), jnp.int32),
        pltpu.VMEM((_NBINS,), jnp.int32),
        pltpu.VMEM((2, 4, 8, _WROWS), jnp.float32),
        pltpu.VMEM((_NTAIL, 128), jnp.float32),
        pltpu.VMEM((_BATCHR, 128), jnp.float32),
        pltpu.VMEM((_BATCHR,), jnp.int32),
        pltpu.SemaphoreType.DMA,
        pltpu.SemaphoreType.DMA,
    ],
    compiler_params=pltpu.CompilerParams(
        use_tc_tiling_on_sc=True, needs_layout_passes=False),
)
def _sweep_kernel(idx_hbm, tab_hbm, tail_hbm, out_hbm,
                  idxbuf, bk_v, bc_v, win_v, tail_v, rows_v, kb_v,
                  sem, osem):
    wid = lax.axis_index("s") * _NC + lax.axis_index("c")
    iota = jax.lax.iota(jnp.int32, 16)
    m_idx = iota & 7
    j_a = jnp.where(iota < 8, 0, 1)
    j_b = j_a + 2
    zeros16 = jnp.zeros((16,), jnp.int32)

    c0 = _c0(wid)
    c1 = jnp.where(wid == _NW - 1, jnp.int32(_TC_FULL), _c0(wid + 1))
    lo = c0 * 128
    hi = jnp.where(wid == _NW - 1, jnp.int32(NUM_ROWS), c1 * 128)
    kdump = jnp.int32(BATCH) + wid

    pltpu.sync_copy(idx_hbm, idxbuf)
    pltpu.sync_copy(tail_hbm, tail_v)

    bc_v[pl.ds(0, 16)] = zeros16
    bc_v[pl.ds(16, 16)] = zeros16
    for b in range(_NBINS * _CAP // 16):
        bk_v[pl.ds(b * 16, 16)] = zeros16

    for j_big in range(4):
        pltpu.async_copy(
            tab_hbm.at[pl.ds(j_big * 8, 8), pl.ds(c0 * 128, _WROWS)],
            win_v.at[0, j_big], sem)

    def scan_body(blk, cnt):
        def grp(g, cnt):
            p = blk * 128 + g * 16
            ivec = idxbuf[pl.ds(p, 16)]
            msk = (ivec >= lo) & (ivec < hi)
            nm = plsc.all_reduce_population_count(msk)[0]

            def do(cnt):
                wv = jnp.clip((ivec - lo) >> 10, 0, _NBINS - 1)
                wv = jnp.where(ivec >= _TAIL0, _NBINS - 1, wv)
                rank, lastm = plsc.scan_count(wv, mask=msk)
                old = plsc.load_gather(bc_v, [wv])
                bpos = old + rank - 1
                plsc.store_scatter(
                    bk_v, [wv * _CAP + jnp.minimum(bpos, _CAP - 1)],
                    p + iota, mask=msk & (bpos < _CAP))
                plsc.store_scatter(bc_v, [wv], old + rank, mask=msk & lastm)
                return cnt + nm
            return lax.cond(nm > 0, do, lambda c: c, cnt)
        for g in range(8):
            cnt = grp(g, cnt)
        return cnt
    cnt = lax.fori_loop(0, BATCH // 128, scan_body, jnp.int32(0))
    del cnt

    def prefill():
        for b in range(_BATCHR // 16):
            kb_v[pl.ds(b * 16, 16)] = jnp.full((16,), kdump, jnp.int32)
    prefill()

    def flush():
        pltpu.async_copy(rows_v, out_hbm.at[kb_v], osem).wait()
        prefill()

    def extract_block(slot, kvec, valid, wlo, is_tail, win_ref=None):
        ivec = plsc.load_gather(idxbuf, [kvec])
        i_sel = jnp.where(valid, ivec, wlo)
        k_sel = jnp.where(valid, kvec, kdump)
        valid_i = valid.astype(jnp.int32)
        for m in range(16):
            pred = valid_i[m] > 0
            i_s = i_sel[m]
            k_s = k_sel[m]

            @pl.when(pred)
            def _():
                cl = i_s - wlo
                if is_tail:
                    r_spl = jnp.full((16,), cl, jnp.int32)
                    ga = plsc.load_gather(tail_v, [r_spl, iota])
                    gb = plsc.load_gather(tail_v, [r_spl, iota + 16])
                else:
                    cl_spl = jnp.full((16,), cl, jnp.int32)
                    ga = plsc.load_gather(win_ref, [j_a, m_idx, cl_spl])
                    gb = plsc.load_gather(win_ref, [j_b, m_idx, cl_spl])
                rows_v[slot, pl.ds(0, 16)] = ga
                rows_v[slot, pl.ds(16, 16)] = gb
                plsc.store_scatter(
                    kb_v, [jnp.full((16,), slot, jnp.int32)],
                    jnp.full((16,), k_s, jnp.int32), mask=iota == 0)
            slot = jnp.where(pred, slot + 1, slot)

        @pl.when(slot > _FLUSH_AT)
        def _():
            flush()
        return jnp.where(slot > _FLUSH_AT, 0, slot)

    def bucket_pass(slot, jw, wlo, bcnt, is_tail, win_ref):
        def grp_body(gg, slot):
            kvec = bk_v[pl.ds(jw * _CAP + gg * 16, 16)]
            valid = gg * 16 + iota < bcnt
            return extract_block(slot, kvec, valid, wlo, is_tail, win_ref)
        return lax.fori_loop(0, (bcnt + 15) // 16, grp_body, slot)

    def rescan_pass(slot, wlo, whi, is_tail, win_ref):
        def grp_body(g, slot):
            base = g * 16
            kvec = base + iota
            ivec = idxbuf[pl.ds(base, 16)]
            valid = (ivec >= wlo) & (ivec < whi)
            nmatch = plsc.all_reduce_population_count(valid)[0]
            return lax.cond(
                nmatch > 0,
                lambda s: extract_block(s, kvec, valid, wlo, is_tail, win_ref),
                lambda s: s, slot)
        return lax.fori_loop(0, BATCH // 16, grp_body, slot)

    def process_bin(slot, jw, wlo, whi, is_tail, win_ref):
        bcnt = plsc.load_gather(bc_v, [jnp.full((16,), jw, jnp.int32)])[0]
        return lax.cond(
            bcnt > _CAP,
            lambda s: rescan_pass(s, wlo, whi, is_tail, win_ref),
            lambda s: bucket_pass(s, jw, wlo, bcnt, is_tail, win_ref), slot)

    nwin = (c1 - c0 + _WCOLS - 1) // _WCOLS

    def wc0_of(jw):
        return jnp.minimum(c0 + jw * _WCOLS, c1 - _WCOLS)

    def issue(jw, buf):
        wc0 = wc0_of(jw)
        for j_big in range(4):
            pltpu.async_copy(
                tab_hbm.at[pl.ds(j_big * 8, 8), pl.ds(wc0 * 128, _WROWS)],
                win_v.at[buf, j_big], sem)

    def wait_win(buf):
        for j_big in range(4):
            pltpu.make_async_copy(
                tab_hbm.at[pl.ds(0, 8), pl.ds(0, _WROWS)],
                win_v.at[buf, j_big], sem).wait()

    def win_body(jw, slot):
        buf = jw & 1
        wait_win(buf)

        @pl.when(jw + 1 < nwin)
        def _():
            issue(jw + 1, buf ^ 1)
        wc0 = wc0_of(jw)
        return process_bin(slot, jw, wc0 * 128, (wc0 + _WCOLS) * 128, False,
                           win_v.at[buf])

    slot = lax.fori_loop(0, nwin, win_body, jnp.int32(0))

    slot = process_bin(slot, jnp.int32(_NBINS - 1), jnp.int32(_TAIL0),
                       jnp.int32(NUM_ROWS), True, win_v.at[0])

    flush()


def kernel(idx_list, table):
    idx = jnp.asarray(idx_list, jnp.int32)
    tab_t = table.T
    tail = jnp.pad(table[_TAIL0:], ((0, 0), (0, 128 - DIM)))
    out_raw = _sweep_kernel(idx, tab_t, tail)
    return out_raw[:BATCH, :DIM].reshape(1, BATCH, DIM)
